# baseline (speedup 1.0000x reference)
import functools

import jax
import jax.numpy as jnp
from jax.experimental import pallas as pl
from jax.experimental.pallas import tpu as pltpu

_VMEM = 64 * 1024 * 1024


def _cgemm_kernel(a_ref, b_ref, bias_ref, o_ref, acc_ref, *, relu):
    t = pl.program_id(1)

    @pl.when(t == 0)
    def _():
        acc_ref[...] = jnp.zeros_like(acc_ref)

    acc_ref[...] += jnp.dot(a_ref[0], b_ref[0],
                            preferred_element_type=jnp.float32)

    @pl.when(t == pl.num_programs(1) - 1)
    def _():
        r = acc_ref[...] + bias_ref[...]
        if relu:
            r = jnp.maximum(r, 0.0)
        o_ref[...] = r


def _conv_gemm(p3, w3, bias, tm, *, relu=False):
    T, M, C = p3.shape
    Cout = w3.shape[2]
    return pl.pallas_call(
        functools.partial(_cgemm_kernel, relu=relu),
        out_shape=jax.ShapeDtypeStruct((M, Cout), jnp.float32),
        grid=(M // tm, T),
        in_specs=[
            pl.BlockSpec((1, tm, C), lambda m, t: (t, m, 0)),
            pl.BlockSpec((1, C, Cout), lambda m, t: (t, 0, 0)),
            pl.BlockSpec((1, Cout), lambda m, t: (0, 0)),
        ],
        out_specs=pl.BlockSpec((tm, Cout), lambda m, t: (m, 0)),
        scratch_shapes=[pltpu.VMEM((tm, Cout), jnp.float32)],
        compiler_params=pltpu.CompilerParams(
            dimension_semantics=("parallel", "arbitrary"),
            vmem_limit_bytes=_VMEM),
    )(p3, w3, bias)


def _c1_kernel(x_ref, w_ref, b_ref, o_ref, acc_ref):
    for di in range(7):
        qh, ph = di // 4, di % 4
        slab = jnp.concatenate(
            [x_ref[0, qh:qh + 160, ph, (dj // 4):(dj // 4) + 109,
                   (dj % 4) * 9:(dj % 4) * 9 + 9]
             for dj in range(7)], axis=2)
        r = jax.lax.dot_general(slab, w_ref[di], (((2,), (0,)), ((), ())),
                                preferred_element_type=jnp.float32)
        if di == 0:
            acc_ref[...] = r
        else:
            acc_ref[...] += r
    o_ref[0] = (acc_ref[...] + b_ref[0][None, None, :]).astype(o_ref.dtype)


def _conv1_fused(x, w, b):
    N = x.shape[0]
    xv = x.reshape(N, 161, 4, 110, 36)
    w3 = jnp.transpose(w, (2, 3, 1, 0)).reshape(7, 63, 24).astype(jnp.bfloat16)
    return pl.pallas_call(
        _c1_kernel,
        out_shape=jax.ShapeDtypeStruct((N, 160, 109, 24), jnp.bfloat16),
        grid=(N,),
        in_specs=[
            pl.BlockSpec((1, 161, 4, 110, 36), lambda n: (n, 0, 0, 0, 0)),
            pl.BlockSpec((7, 63, 24), lambda n: (0, 0, 0)),
            pl.BlockSpec((1, 24), lambda n: (0, 0)),
        ],
        out_specs=pl.BlockSpec((1, 160, 109, 24), lambda n: (n, 0, 0, 0)),
        scratch_shapes=[pltpu.VMEM((160, 109, 24), jnp.float32)],
        compiler_params=pltpu.CompilerParams(
            dimension_semantics=("parallel",), vmem_limit_bytes=_VMEM),
    )(xv, w3, b.astype(jnp.float32).reshape(1, 24))


def _patches3(x, kh, kw, s, Ho, Wo):
    N, _, _, C = x.shape
    slabs = [x[:, di:di + s * (Ho - 1) + 1:s, dj:dj + s * (Wo - 1) + 1:s, :]
             .reshape(1, N * Ho * Wo, C)
             for di in range(kh) for dj in range(kw)]
    return jnp.concatenate(slabs, axis=0)


def _wtap(w, scale=None):
    Cout, Cin = w.shape[0], w.shape[1]
    wm = jnp.transpose(w, (2, 3, 1, 0))
    if scale is not None:
        wm = wm * scale[None, None, None, :]
    return wm.reshape(-1, Cin, Cout).astype(jnp.bfloat16)


def _gemm1_kernel(a_ref, b_ref, bias_ref, o_ref, *, relu):
    r = jnp.dot(a_ref[...], b_ref[...], preferred_element_type=jnp.float32)
    r = r + bias_ref[...]
    if relu:
        r = jnp.maximum(r, 0.0)
    o_ref[...] = r.astype(o_ref.dtype)


def _gemm1(a, b, bias, tm, *, relu=False, parallel=True):
    M, K = a.shape
    N = b.shape[1]
    nb = M // tm
    sem = ("parallel",) if parallel and nb > 1 else ("arbitrary",)
    return pl.pallas_call(
        functools.partial(_gemm1_kernel, relu=relu),
        out_shape=jax.ShapeDtypeStruct((M, N), jnp.float32),
        grid=(nb,),
        in_specs=[
            pl.BlockSpec((tm, K), lambda i: (i, 0)),
            pl.BlockSpec((K, N), lambda i: (0, 0)),
            pl.BlockSpec((1, N), lambda i: (0, 0)),
        ],
        out_specs=pl.BlockSpec((tm, N), lambda i: (i, 0)),
        compiler_params=pltpu.CompilerParams(
            dimension_semantics=sem, vmem_limit_bytes=_VMEM),
    )(a, b, bias)


def _lin0_kernel(a_ref, b_ref, bias_ref, o_ref, acc_ref):
    k = pl.program_id(1)

    @pl.when(k == 0)
    def _():
        acc_ref[...] = jnp.zeros_like(acc_ref)

    acc_ref[...] += jnp.dot(a_ref[...], b_ref[...],
                            preferred_element_type=jnp.float32)

    @pl.when(k == pl.num_programs(1) - 1)
    def _():
        o_ref[...] = jnp.maximum(acc_ref[...] + bias_ref[...], 0.0)


def _linear0(x, w, b):
    M, K = x.shape
    N = w.shape[1]
    tn, tk = N // 2, 2048
    out = pl.pallas_call(
        _lin0_kernel,
        out_shape=jax.ShapeDtypeStruct((M, N), jnp.float32),
        grid=(N // tn, K // tk),
        in_specs=[
            pl.BlockSpec((M, tk), lambda j, k: (0, k)),
            pl.BlockSpec((tk, tn), lambda j, k: (k, j)),
            pl.BlockSpec((1, tn), lambda j, k: (0, j)),
        ],
        out_specs=pl.BlockSpec((M, tn), lambda j, k: (0, j)),
        scratch_shapes=[pltpu.VMEM((M, tn), jnp.float32)],
        compiler_params=pltpu.CompilerParams(
            dimension_semantics=("parallel", "arbitrary"), vmem_limit_bytes=_VMEM),
    )(x.astype(jnp.bfloat16), w.astype(jnp.bfloat16),
      b.astype(jnp.float32).reshape(1, N))
    return out


def _pool_kernel(x_ref, sc_ref, sh_ref, o_ref, *, C, Ho, Wo):
    x = x_ref[...]
    woc = Wo * C
    rw = jnp.maximum(jnp.maximum(x[:, :, :woc], x[:, :, C:woc + C]),
                     x[:, :, 2 * C:woc + 2 * C])
    m = jnp.maximum(jnp.maximum(rw[:, 0:Ho], rw[:, 1:Ho + 1]), rw[:, 2:Ho + 2])
    o_ref[...] = jnp.maximum(m * sc_ref[...][:, None, :] + sh_ref[...][:, None, :], 0.0)


def _pool_bn_relu(x, scale, shift, Hin):
    N, Hp, W, C = x.shape
    Ho, Wo = Hin - 2, W - 2
    sc = jnp.tile(scale.astype(jnp.float32), Wo).reshape(1, Wo * C)
    sh = jnp.tile(shift.astype(jnp.float32), Wo).reshape(1, Wo * C)
    out = pl.pallas_call(
        functools.partial(_pool_kernel, C=C, Ho=Ho, Wo=Wo),
        out_shape=jax.ShapeDtypeStruct((N, Ho, Wo * C), jnp.float32),
        grid=(N,),
        in_specs=[
            pl.BlockSpec((1, Hp, W * C), lambda n: (n, 0, 0)),
            pl.BlockSpec((1, Wo * C), lambda n: (0, 0)),
            pl.BlockSpec((1, Wo * C), lambda n: (0, 0)),
        ],
        out_specs=pl.BlockSpec((1, Ho, Wo * C), lambda n: (n, 0, 0)),
        compiler_params=pltpu.CompilerParams(
            dimension_semantics=("parallel",), vmem_limit_bytes=_VMEM),
    )(x.reshape(N, Hp, W * C), sc, sh)
    return out.reshape(N, Ho, Wo, C)


def _tail_kernel(x_ref, w1, b1, w2, b2, w3, b3, w4, b4, w5, b5, o_ref):
    h = x_ref[...]
    for w, b in ((w1, b1), (w2, b2), (w3, b3), (w4, b4)):
        h = jnp.maximum(
            jnp.dot(h, w[...], preferred_element_type=jnp.float32) + b[...], 0.0)
    o_ref[...] = jnp.dot(h, w5[...], preferred_element_type=jnp.float32) + b5[...]


def _fc_tail(x, ws):
    M = x.shape[0]

    def pad(w, b, ki, ko):
        wp = jnp.pad(w.astype(jnp.float32), ((0, ki - w.shape[0]), (0, ko - w.shape[1])))
        bp = jnp.pad(b.astype(jnp.float32), (0, ko - b.shape[0])).reshape(1, ko)
        return wp, bp

    (w1, b1), (w2, b2), (w3, b3), (w4, b4), (w5, b5) = ws
    args = (x,) + pad(w1, b1, 1024, 512) + pad(w2, b2, 512, 128) + \
        pad(w3, b3, 128, 128) + pad(w4, b4, 128, 128) + pad(w5, b5, 128, 7)
    return pl.pallas_call(
        _tail_kernel,
        out_shape=jax.ShapeDtypeStruct((M, 7), jnp.float32),
        grid=(1,),
        in_specs=[pl.BlockSpec(a.shape, lambda i: (0,) * a.ndim) for a in args],
        out_specs=pl.BlockSpec((M, 7), lambda i: (0, 0)),
        compiler_params=pltpu.CompilerParams(
            dimension_semantics=("arbitrary",), vmem_limit_bytes=_VMEM),
    )(*args)


def _wmat(w, Kpad, scale=None):
    Cout = w.shape[0]
    wm = jnp.transpose(w, (2, 3, 1, 0)).reshape(-1, Cout)
    if scale is not None:
        wm = wm * scale[None, :]
    K = wm.shape[0]
    if Kpad > K:
        wm = jnp.pad(wm, ((0, Kpad - K), (0, 0)))
    return wm.astype(jnp.bfloat16)


def _bn_ss(gamma, beta, mean, var):
    scl = gamma * jax.lax.rsqrt(var + 1e-5)
    return scl, beta - mean * scl


def kernel(x_f, x_l, x_r, conv1_w, conv1_b, conv2_w, conv2_b, conv3_w, conv3_b,
           conv4_w, conv4_b, conv5_w, conv5_b,
           bn1_gamma, bn1_beta, bn1_mean, bn1_var,
           bn2_gamma, bn2_beta, bn2_mean, bn2_var,
           bn3_gamma, bn3_beta, bn3_mean, bn3_var,
           bn4_gamma, bn4_beta, bn4_mean, bn4_var,
           bn5_gamma, bn5_beta, bn5_mean, bn5_var,
           linear0_w, linear0_b, linear1_w, linear1_b, linear2_w, linear2_b,
           linear3_w, linear3_b, linear4_w, linear4_b, linear5_w, linear5_b):
    N = x_f.shape[0]

    x = jnp.concatenate([x_f, x_l, x_r], axis=1).astype(jnp.bfloat16)
    x = jnp.transpose(x, (0, 2, 3, 1))
    x = jnp.pad(x, ((0, 0), (0, 13), (0, 1), (0, 0)))

    y = _conv1_fused(x, conv1_w, conv1_b)
    s, h = _bn_ss(bn1_gamma, bn1_beta, bn1_mean, bn1_var)
    y = _pool_bn_relu(y, s, h, 157)

    yb = jnp.pad(y.astype(jnp.bfloat16), ((0, 0), (0, 16), (0, 0), (0, 0)))
    p = _patches3(yb, 5, 5, 3, 56, 35)
    y = _conv_gemm(p, _wtap(conv2_w),
                   conv2_b.astype(jnp.float32).reshape(1, 36), tm=1960)
    y = y.reshape(N, 56, 35, 36)
    s, h = _bn_ss(bn2_gamma, bn2_beta, bn2_mean, bn2_var)
    y = _pool_bn_relu(y, s, h, 51)

    yb = jnp.pad(y.astype(jnp.bfloat16), ((0, 0), (0, 2), (0, 0), (0, 0)))
    p = _patches3(yb, 5, 5, 2, 24, 15)
    y = _conv_gemm(p, _wtap(conv3_w),
                   conv3_b.astype(jnp.float32).reshape(1, 48), tm=1440)
    y = y.reshape(N, 24, 15, 48)
    s, h = _bn_ss(bn3_gamma, bn3_beta, bn3_mean, bn3_var)
    y = _pool_bn_relu(y, s, h, 23)

    yb = y.astype(jnp.bfloat16)
    p = _patches3(yb, 3, 3, 2, 10, 6)
    y = _conv_gemm(p, _wtap(conv4_w),
                   conv4_b.astype(jnp.float32).reshape(1, 256), tm=480)
    y = y.reshape(N, 10, 6, 256)
    s, h = _bn_ss(bn4_gamma, bn4_beta, bn4_mean, bn4_var)
    y = _pool_bn_relu(y, s, h, 10)

    s, h = _bn_ss(bn5_gamma, bn5_beta, bn5_mean, bn5_var)
    b5 = (conv5_b * s + h).astype(jnp.float32).reshape(1, 768)
    p = y.astype(jnp.bfloat16).reshape(N * 32, 256)
    y = _gemm1(p, _wmat(conv5_w, 256, scale=s), b5, tm=N * 32,
               parallel=False, relu=True)

    y = jnp.transpose(y.reshape(N, 32, 768), (0, 2, 1)).reshape(N, 24576)

    y = _linear0(y, linear0_w, linear0_b)
    return _fc_tail(y, ((linear1_w, linear1_b), (linear2_w, linear2_b),
                        (linear3_w, linear3_b), (linear4_w, linear4_b),
                        (linear5_w, linear5_b)))


# speedup vs baseline: 1.2190x; 1.2190x over previous
import functools

import jax
import jax.numpy as jnp
from jax.experimental import pallas as pl
from jax.experimental.pallas import tpu as pltpu

_VMEM = 64 * 1024 * 1024


def _cgemm_kernel(a_ref, b_ref, bias_ref, o_ref, acc_ref, *, relu):
    t = pl.program_id(1)

    @pl.when(t == 0)
    def _():
        acc_ref[...] = jnp.zeros_like(acc_ref)

    acc_ref[...] += jnp.dot(a_ref[0], b_ref[0],
                            preferred_element_type=jnp.float32)

    @pl.when(t == pl.num_programs(1) - 1)
    def _():
        r = acc_ref[...] + bias_ref[...]
        if relu:
            r = jnp.maximum(r, 0.0)
        o_ref[...] = r


def _conv_gemm(p3, w3, bias, tm, *, relu=False):
    T, M, C = p3.shape
    Cout = w3.shape[2]
    return pl.pallas_call(
        functools.partial(_cgemm_kernel, relu=relu),
        out_shape=jax.ShapeDtypeStruct((M, Cout), jnp.float32),
        grid=(M // tm, T),
        in_specs=[
            pl.BlockSpec((1, tm, C), lambda m, t: (t, m, 0)),
            pl.BlockSpec((1, C, Cout), lambda m, t: (t, 0, 0)),
            pl.BlockSpec((1, Cout), lambda m, t: (0, 0)),
        ],
        out_specs=pl.BlockSpec((tm, Cout), lambda m, t: (m, 0)),
        scratch_shapes=[pltpu.VMEM((tm, Cout), jnp.float32)],
        compiler_params=pltpu.CompilerParams(
            dimension_semantics=("parallel", "arbitrary"),
            vmem_limit_bytes=_VMEM),
    )(p3, w3, bias)


def _patches3(x, kh, kw, s, Ho, Wo):
    N, _, _, C = x.shape
    slabs = [x[:, di:di + s * (Ho - 1) + 1:s, dj:dj + s * (Wo - 1) + 1:s, :]
             .reshape(1, N * Ho * Wo, C)
             for di in range(kh) for dj in range(kw)]
    return jnp.concatenate(slabs, axis=0)


def _wtap(w, scale=None):
    Cout, Cin = w.shape[0], w.shape[1]
    wm = jnp.transpose(w, (2, 3, 1, 0))
    if scale is not None:
        wm = wm * scale[None, None, None, :]
    return wm.reshape(-1, Cin, Cout).astype(jnp.bfloat16)


def _gemm1_kernel(a_ref, b_ref, bias_ref, o_ref, *, relu):
    r = jnp.dot(a_ref[...], b_ref[...], preferred_element_type=jnp.float32)
    r = r + bias_ref[...]
    if relu:
        r = jnp.maximum(r, 0.0)
    o_ref[...] = r.astype(o_ref.dtype)


def _gemm1(a, b, bias, tm, *, relu=False, parallel=True):
    M, K = a.shape
    N = b.shape[1]
    nb = M // tm
    sem = ("parallel",) if parallel and nb > 1 else ("arbitrary",)
    return pl.pallas_call(
        functools.partial(_gemm1_kernel, relu=relu),
        out_shape=jax.ShapeDtypeStruct((M, N), jnp.float32),
        grid=(nb,),
        in_specs=[
            pl.BlockSpec((tm, K), lambda i: (i, 0)),
            pl.BlockSpec((K, N), lambda i: (0, 0)),
            pl.BlockSpec((1, N), lambda i: (0, 0)),
        ],
        out_specs=pl.BlockSpec((tm, N), lambda i: (i, 0)),
        compiler_params=pltpu.CompilerParams(
            dimension_semantics=sem, vmem_limit_bytes=_VMEM),
    )(a, b, bias)


def _lin0_kernel(a_ref, b_ref, bias_ref, o_ref, acc_ref):
    k = pl.program_id(1)

    @pl.when(k == 0)
    def _():
        acc_ref[...] = jnp.zeros_like(acc_ref)

    acc_ref[...] += jnp.dot(a_ref[...], b_ref[...],
                            preferred_element_type=jnp.float32)

    @pl.when(k == pl.num_programs(1) - 1)
    def _():
        o_ref[...] = jnp.maximum(acc_ref[...] + bias_ref[...], 0.0)


def _linear0(x, w, b):
    M, K = x.shape
    N = w.shape[1]
    tn, tk = N // 2, 2048
    out = pl.pallas_call(
        _lin0_kernel,
        out_shape=jax.ShapeDtypeStruct((M, N), jnp.float32),
        grid=(N // tn, K // tk),
        in_specs=[
            pl.BlockSpec((M, tk), lambda j, k: (0, k)),
            pl.BlockSpec((tk, tn), lambda j, k: (k, j)),
            pl.BlockSpec((1, tn), lambda j, k: (0, j)),
        ],
        out_specs=pl.BlockSpec((M, tn), lambda j, k: (0, j)),
        scratch_shapes=[pltpu.VMEM((M, tn), jnp.float32)],
        compiler_params=pltpu.CompilerParams(
            dimension_semantics=("parallel", "arbitrary"), vmem_limit_bytes=_VMEM),
    )(x.astype(jnp.bfloat16), w.astype(jnp.bfloat16),
      b.astype(jnp.float32).reshape(1, N))
    return out


def _pool_kernel(x_ref, sc_ref, sh_ref, o_ref, *, C, Ho, Wo):
    x = x_ref[...]
    woc = Wo * C
    rw = jnp.maximum(jnp.maximum(x[:, :, :woc], x[:, :, C:woc + C]),
                     x[:, :, 2 * C:woc + 2 * C])
    m = jnp.maximum(jnp.maximum(rw[:, 0:Ho], rw[:, 1:Ho + 1]), rw[:, 2:Ho + 2])
    o_ref[...] = jnp.maximum(m * sc_ref[...][:, None, :] + sh_ref[...][:, None, :], 0.0)


def _pool_bn_relu(x, scale, shift, Hin):
    N, Hp, W, C = x.shape
    Ho, Wo = Hin - 2, W - 2
    sc = jnp.tile(scale.astype(jnp.float32), Wo).reshape(1, Wo * C)
    sh = jnp.tile(shift.astype(jnp.float32), Wo).reshape(1, Wo * C)
    out = pl.pallas_call(
        functools.partial(_pool_kernel, C=C, Ho=Ho, Wo=Wo),
        out_shape=jax.ShapeDtypeStruct((N, Ho, Wo * C), jnp.float32),
        grid=(N,),
        in_specs=[
            pl.BlockSpec((1, Hp, W * C), lambda n: (n, 0, 0)),
            pl.BlockSpec((1, Wo * C), lambda n: (0, 0)),
            pl.BlockSpec((1, Wo * C), lambda n: (0, 0)),
        ],
        out_specs=pl.BlockSpec((1, Ho, Wo * C), lambda n: (n, 0, 0)),
        compiler_params=pltpu.CompilerParams(
            dimension_semantics=("parallel",), vmem_limit_bytes=_VMEM),
    )(x.reshape(N, Hp, W * C), sc, sh)
    return out.reshape(N, Ho, Wo, C)


def _tail_kernel(x_ref, w1, b1, w2, b2, w3, b3, w4, b4, w5, b5, o_ref):
    h = x_ref[...]
    for w, b in ((w1, b1), (w2, b2), (w3, b3), (w4, b4)):
        h = jnp.maximum(
            jnp.dot(h, w[...], preferred_element_type=jnp.float32) + b[...], 0.0)
    o_ref[...] = jnp.dot(h, w5[...], preferred_element_type=jnp.float32) + b5[...]


def _fc_tail(x, ws):
    M = x.shape[0]

    def pad(w, b, ki, ko):
        wp = jnp.pad(w.astype(jnp.float32), ((0, ki - w.shape[0]), (0, ko - w.shape[1])))
        bp = jnp.pad(b.astype(jnp.float32), (0, ko - b.shape[0])).reshape(1, ko)
        return wp, bp

    (w1, b1), (w2, b2), (w3, b3), (w4, b4), (w5, b5) = ws
    args = (x,) + pad(w1, b1, 1024, 512) + pad(w2, b2, 512, 128) + \
        pad(w3, b3, 128, 128) + pad(w4, b4, 128, 128) + pad(w5, b5, 128, 7)
    return pl.pallas_call(
        _tail_kernel,
        out_shape=jax.ShapeDtypeStruct((M, 7), jnp.float32),
        grid=(1,),
        in_specs=[pl.BlockSpec(a.shape, lambda i: (0,) * a.ndim) for a in args],
        out_specs=pl.BlockSpec((M, 7), lambda i: (0, 0)),
        compiler_params=pltpu.CompilerParams(
            dimension_semantics=("arbitrary",), vmem_limit_bytes=_VMEM),
    )(*args)


def _wmat(w, Kpad, scale=None):
    Cout = w.shape[0]
    wm = jnp.transpose(w, (2, 3, 1, 0)).reshape(-1, Cout)
    if scale is not None:
        wm = wm * scale[None, :]
    K = wm.shape[0]
    if Kpad > K:
        wm = jnp.pad(wm, ((0, Kpad - K), (0, 0)))
    return wm.astype(jnp.bfloat16)


def _bn_ss(gamma, beta, mean, var):
    scl = gamma * jax.lax.rsqrt(var + 1e-5)
    return scl, beta - mean * scl


def kernel(x_f, x_l, x_r, conv1_w, conv1_b, conv2_w, conv2_b, conv3_w, conv3_b,
           conv4_w, conv4_b, conv5_w, conv5_b,
           bn1_gamma, bn1_beta, bn1_mean, bn1_var,
           bn2_gamma, bn2_beta, bn2_mean, bn2_var,
           bn3_gamma, bn3_beta, bn3_mean, bn3_var,
           bn4_gamma, bn4_beta, bn4_mean, bn4_var,
           bn5_gamma, bn5_beta, bn5_mean, bn5_var,
           linear0_w, linear0_b, linear1_w, linear1_b, linear2_w, linear2_b,
           linear3_w, linear3_b, linear4_w, linear4_b, linear5_w, linear5_b):
    N = x_f.shape[0]

    x = jnp.concatenate([x_f, x_l, x_r], axis=1).astype(jnp.bfloat16)
    x = jnp.transpose(x, (0, 2, 3, 1))
    x = jnp.pad(x, ((0, 0), (0, 12), (0, 0), (0, 0)))

    p = _patches3(x, 7, 7, 4, 160, 109)
    y = _conv_gemm(p, _wtap(conv1_w),
                   conv1_b.astype(jnp.float32).reshape(1, 24), tm=1744)
    y = y.reshape(N, 160, 109, 24)
    s, h = _bn_ss(bn1_gamma, bn1_beta, bn1_mean, bn1_var)
    y = _pool_bn_relu(y, s, h, 157)

    yb = jnp.pad(y.astype(jnp.bfloat16), ((0, 0), (0, 16), (0, 0), (0, 0)))
    p = _patches3(yb, 5, 5, 3, 56, 35)
    y = _conv_gemm(p, _wtap(conv2_w),
                   conv2_b.astype(jnp.float32).reshape(1, 36), tm=1960)
    y = y.reshape(N, 56, 35, 36)
    s, h = _bn_ss(bn2_gamma, bn2_beta, bn2_mean, bn2_var)
    y = _pool_bn_relu(y, s, h, 51)

    yb = jnp.pad(y.astype(jnp.bfloat16), ((0, 0), (0, 2), (0, 0), (0, 0)))
    p = _patches3(yb, 5, 5, 2, 24, 15)
    y = _conv_gemm(p, _wtap(conv3_w),
                   conv3_b.astype(jnp.float32).reshape(1, 48), tm=1440)
    y = y.reshape(N, 24, 15, 48)
    s, h = _bn_ss(bn3_gamma, bn3_beta, bn3_mean, bn3_var)
    y = _pool_bn_relu(y, s, h, 23)

    yb = y.astype(jnp.bfloat16)
    p = _patches3(yb, 3, 3, 2, 10, 6)
    y = _conv_gemm(p, _wtap(conv4_w),
                   conv4_b.astype(jnp.float32).reshape(1, 256), tm=480)
    y = y.reshape(N, 10, 6, 256)
    s, h = _bn_ss(bn4_gamma, bn4_beta, bn4_mean, bn4_var)
    y = _pool_bn_relu(y, s, h, 10)

    s, h = _bn_ss(bn5_gamma, bn5_beta, bn5_mean, bn5_var)
    b5 = (conv5_b * s + h).astype(jnp.float32).reshape(1, 768)
    p = y.astype(jnp.bfloat16).reshape(N * 32, 256)
    y = _gemm1(p, _wmat(conv5_w, 256, scale=s), b5, tm=N * 32,
               parallel=False, relu=True)

    y = jnp.transpose(y.reshape(N, 32, 768), (0, 2, 1)).reshape(N, 24576)

    y = _linear0(y, linear0_w, linear0_b)
    return _fc_tail(y, ((linear1_w, linear1_b), (linear2_w, linear2_b),
                        (linear3_w, linear3_b), (linear4_w, linear4_b),
                        (linear5_w, linear5_b)))
